# baseline (speedup 1.0000x reference)
import jax
import jax.numpy as jnp
from jax import lax
from jax.experimental import pallas as pl
from jax.experimental.pallas import tpu as pltpu

M_ROWS = 1_000_000
DIM = 16
NJ = 114688


def _tc_body(u_ref, x_ref, o_ref):
    res = lax.dot_general(
        u_ref[...],
        x_ref[...],
        (((1,), (0,)), ((), ())),
        preferred_element_type=jnp.float32,
    )
    o_ref[...] = res.reshape(-1)


def kernel(user_emb, items_emb):
    items_t = items_emb.T
    u2 = user_emb.reshape(1, DIM)
    grid = (pl.cdiv(M_ROWS, NJ),)
    return pl.pallas_call(
        _tc_body,
        grid=grid,
        in_specs=[
            pl.BlockSpec((1, DIM), lambda i: (0, 0)),
            pl.BlockSpec((DIM, NJ), lambda i: (0, i)),
        ],
        out_specs=pl.BlockSpec((NJ,), lambda i: (i,)),
        out_shape=jax.ShapeDtypeStruct((M_ROWS,), jnp.float32),
        compiler_params=pltpu.CompilerParams(
            dimension_semantics=("parallel",),
        ),
    )(u2, items_t)


# speedup vs baseline: 1.0046x; 1.0046x over previous
import jax
import jax.numpy as jnp
from jax import lax
from jax.experimental import pallas as pl
from jax.experimental.pallas import tpu as pltpu

M_ROWS = 1_000_000
DIM = 16
NJ = 114688


def _tc_body(u_ref, x_ref, o_ref):
    res = lax.dot_general(
        u_ref[...],
        x_ref[...],
        (((1,), (0,)), ((), ())),
        preferred_element_type=jnp.float32,
    )
    o_ref[...] = res.reshape(-1)


def kernel(user_emb, items_emb):
    items_t = items_emb.T
    u2 = user_emb.reshape(1, DIM)
    grid = (pl.cdiv(M_ROWS, NJ),)
    return pl.pallas_call(
        _tc_body,
        grid=grid,
        in_specs=[
            pl.BlockSpec((1, DIM), lambda i: (0, 0)),
            pl.BlockSpec((DIM, NJ), lambda i: (0, i)),
        ],
        out_specs=pl.BlockSpec((NJ,), lambda i: (i,)),
        out_shape=jax.ShapeDtypeStruct((M_ROWS,), jnp.float32),
        compiler_params=pltpu.CompilerParams(
            dimension_semantics=("arbitrary",),
        ),
    )(u2, items_t)
